# baseline (speedup 1.0000x reference)
import functools

import jax
import jax.numpy as jnp
from jax import lax
from jax.experimental import pallas as pl
from jax.experimental.pallas import tpu as pltpu
from jax.experimental.pallas import tpu_sc as plsc

_DIM = 64
_SCALE = 8.0
_LANES = 16
_CHUNK = 256


@functools.lru_cache(maxsize=None)
def _make_kernel(batch_flat: int):
    info = plsc.get_sparse_core_info()
    nc, ns = info.num_cores, info.num_subcores
    nw = nc * ns
    assert batch_flat % nw == 0
    b_per_w = batch_flat // nw
    chunk = _CHUNK
    assert b_per_w % (2 * chunk) == 0
    n_chunks = b_per_w // chunk
    vecs_per_row = _DIM // _LANES

    mesh = plsc.VectorSubcoreMesh(core_axis_name="c", subcore_axis_name="s")

    @functools.partial(
        pl.kernel,
        mesh=mesh,
        out_type=jax.ShapeDtypeStruct((batch_flat * _DIM // 128, 128), jnp.float32),
        scratch_types=[
            pltpu.VMEM((b_per_w,), jnp.int32),
            pltpu.VMEM((2, chunk, 128), jnp.float32),
            pltpu.VMEM((chunk // 2, 128), jnp.float32),
            pltpu.SemaphoreType.DMA,
            pltpu.SemaphoreType.DMA,
        ],
    )
    def k(ids_hbm, table_hbm, out_hbm, idx_v, rows_v, pack_v, gsem0, gsem1):
        wid = lax.axis_index("s") * nc + lax.axis_index("c")
        base = pl.multiple_of(wid * b_per_w, b_per_w)
        gsems = (gsem0, gsem1)

        pltpu.sync_copy(ids_hbm.at[pl.ds(base, b_per_w)], idx_v)
        pltpu.async_copy(
            table_hbm.at[idx_v.at[pl.ds(0, chunk)]], rows_v.at[0], gsem0
        )

        def pair_body(p, carry):
            for b in range(2):
                g = 2 * p + b
                nb = 1 - b

                @pl.when(g + 1 < n_chunks)
                def _start_next():
                    pltpu.async_copy(
                        table_hbm.at[
                            idx_v.at[
                                pl.ds(pl.multiple_of((g + 1) * chunk, chunk), chunk)
                            ]
                        ],
                        rows_v.at[nb],
                        gsems[nb],
                    )

                pltpu.make_async_copy(
                    table_hbm.at[idx_v.at[pl.ds(0, chunk)]],
                    rows_v.at[b],
                    gsems[b],
                ).wait()

                gbuf = rows_v.at[b]

                @plsc.parallel_loop(0, chunk, unroll=4)
                def _scale_row(r):
                    half = (r % 2) * _DIM
                    for v in range(vecs_per_row):
                        pack_v[r // 2, pl.ds(half + v * _LANES, _LANES)] = (
                            gbuf[r, pl.ds(v * _LANES, _LANES)] * _SCALE
                        )

                pltpu.sync_copy(
                    pack_v,
                    out_hbm.at[
                        pl.ds(
                            pl.multiple_of(
                                (base + g * chunk) * _DIM // 128,
                                chunk * _DIM // 128,
                            ),
                            chunk * _DIM // 128,
                        )
                    ],
                )
            return carry

        lax.fori_loop(0, n_chunks // 2, pair_body, 0)

    return k


def kernel(input_ids, table):
    b, h = input_ids.shape
    flat_ids = input_ids.reshape(b * h).astype(jnp.int32)
    table128 = jnp.concatenate([table, table], axis=1)
    packed = _make_kernel(b * h)(flat_ids, table128)
    return packed.reshape(b, h, _DIM)
